# baseline (speedup 1.0000x reference)
import functools

import jax
import jax.numpy as jnp
from jax import lax
from jax.experimental import pallas as pl
from jax.experimental.pallas import tpu as pltpu
from jax.experimental.pallas import tpu_sc as plsc

NUM_PEAKS = 1000000
HIDDEN = 64
NUM_TOPICS = 32
B = 4096
L = 200

NC = 2
NS = 16
NW = NC * NS
PER_W = B // NW
CHUNK = 100
NCHUNK = L // CHUNK


NBUF = 4
UN = 8
NG = HIDDEN // 16


SLAB_R = 216
IDXW = 112
CHUNK_D = 104
LR = NCHUNK * CHUNK_D


def _sc_pool_body(idxT_hbm, table_hbm, out_hbm, slab_v, idx_v, rows_v,
                  pool_v, s0, s1, s2, s3):
    sems = (s0, s1, s2, s3)
    wid = lax.axis_index("s") * NC + lax.axis_index("c")
    pltpu.sync_copy(idxT_hbm.at[:, pl.ds(wid * PER_W, PER_W)],
                    slab_v.at[pl.ds(0, L)])
    lanes = lax.iota(jnp.int32, 16)

    def prep(e, _):
        col = jnp.full((16,), 0, jnp.int32) + e
        for c in range(NCHUNK):
            for k in range(IDXW // 16):
                rows = CHUNK * c + 16 * k + lanes
                r = plsc.load_gather(slab_v, [rows, col])
                q = r & (TBLK - 1)
                g = (r - q) + 2 * (q & (THALF - 1)) + (q >> 14)
                if 16 * (k + 1) > CHUNK:
                    g = jnp.where(lanes < CHUNK - 16 * k, g, 0)
                idx_v[NCHUNK * e + c, pl.ds(16 * k, 16)] = g
        return 0

    lax.fori_loop(0, PER_W, prep, 0)

    def descs(e, j):
        return (
            pltpu.make_async_copy(
                table_hbm.at[idx_v.at[NCHUNK * e, pl.ds(0, CHUNK_D)]],
                rows_v.at[j, pl.ds(0, CHUNK_D)], sems[j]),
            pltpu.make_async_copy(
                table_hbm.at[idx_v.at[NCHUNK * e + 1, pl.ds(0, CHUNK_D)]],
                rows_v.at[j, pl.ds(CHUNK_D, CHUNK_D)], sems[j]),
        )

    def start(e, j):
        for d in descs(e, j):
            d.start()

    def wait(e, j):
        for d in descs(e, j):
            d.wait()

    def reduce_into(j, b):
        def red(r, acc):
            accs = list(acc)
            for u in range(UN):
                for g in range(NG):
                    a = 2 * g + (u & 1)
                    accs[a] = accs[a] + rows_v[j, r * UN + u,
                                               pl.ds(16 * g, 16)]
            return tuple(accs)

        acc = lax.fori_loop(0, LR // UN, red,
                            tuple(jnp.zeros((16,), jnp.float32)
                                  for _ in range(2 * NG)))
        for g in range(NG):
            pool_v[b, pl.ds(16 * g, 16)] = acc[2 * g] + acc[2 * g + 1]

    for j in range(NBUF):
        start(j, j)

    def outer(i, _):
        k = i * NBUF
        for j in range(NBUF):
            e = k + j
            wait(e, j)
            reduce_into(j, e)

            @pl.when(e + NBUF < PER_W)
            def _():
                start(e + NBUF, j)
        return 0

    lax.fori_loop(0, PER_W // NBUF, outer, 0)
    pltpu.sync_copy(pool_v, out_hbm.at[pl.ds(wid * PER_W, PER_W)])


_sc_pool = functools.partial(
    pl.kernel,
    mesh=plsc.VectorSubcoreMesh(core_axis_name="c", subcore_axis_name="s"),
    out_type=jax.ShapeDtypeStruct((B, HIDDEN), jnp.float32),
    scratch_types=[
        pltpu.VMEM((SLAB_R, PER_W), jnp.int32),
        pltpu.VMEM((PER_W * NCHUNK, IDXW), jnp.int32),
        pltpu.VMEM((NBUF, LR, HIDDEN), jnp.float32),
        pltpu.VMEM((PER_W, HIDDEN), jnp.float32),
        pltpu.SemaphoreType.DMA,
        pltpu.SemaphoreType.DMA,
        pltpu.SemaphoreType.DMA,
        pltpu.SemaphoreType.DMA,
    ],
    compiler_params=pltpu.CompilerParams(use_tc_tiling_on_sc=False,
                                         needs_layout_passes=False),
)(_sc_pool_body)


TBLK = 32768
TGRID = -(-NUM_PEAKS // TBLK)
THALF = TBLK // 2
NP2 = TGRID * TBLK


def _relayout_body(tT_ref, out_ref):
    out_ref[:, 0:HIDDEN] = jnp.transpose(tT_ref[:, 0:THALF])
    out_ref[:, HIDDEN:2 * HIDDEN] = jnp.transpose(tT_ref[:, THALF:TBLK])


def _relayout(tableT):
    return pl.pallas_call(
        _relayout_body,
        grid=(TGRID,),
        in_specs=[pl.BlockSpec((HIDDEN, TBLK), lambda i: (0, i))],
        out_specs=pl.BlockSpec((THALF, 2 * HIDDEN), lambda i: (i, 0)),
        out_shape=jax.ShapeDtypeStruct((NP2 // 2, 2 * HIDDEN),
                                       jnp.float32),
    )(tableT)


def _softplus(x):
    return jnp.maximum(x, 0.0) + jnp.log1p(jnp.exp(-jnp.abs(x)))


def _mlp_body(pooled_ref, rd_ref, W1a_ref, w1b_ref, b1_ref, W2_ref, b2_ref,
              Wmu_ref, bmu_ref, Wlv_ref, blv_ref, gmu_ref, betamu_ref,
              glv_ref, betalv_ref, loc_ref, scale_ref):
    rd = rd_ref[:]
    ave = pooled_ref[:] / rd
    lrd = jnp.log(rd)
    z1 = (lax.dot_general(ave, W1a_ref[:], (((1,), (1,)), ((), ())),
                          preferred_element_type=jnp.float32)
          + lrd * w1b_ref[:] + b1_ref[:])
    h1 = _softplus(z1)
    z2 = (lax.dot_general(h1, W2_ref[:], (((1,), (1,)), ((), ())),
                          preferred_element_type=jnp.float32)
          + b2_ref[:])
    h2 = _softplus(z2)

    def bn(z, g, bt):
        m = jnp.mean(z, axis=0)
        v = jnp.mean((z - m) ** 2, axis=0)
        return (z - m) / jnp.sqrt(v + 1e-5) * g + bt

    zmu = (lax.dot_general(h2, Wmu_ref[:], (((1,), (1,)), ((), ())),
                           preferred_element_type=jnp.float32)
           + bmu_ref[:])
    zlv = (lax.dot_general(h2, Wlv_ref[:], (((1,), (1,)), ((), ())),
                           preferred_element_type=jnp.float32)
           + blv_ref[:])
    loc_ref[:] = bn(zmu, gmu_ref[:], betamu_ref[:])
    scale_ref[:] = jnp.exp(0.5 * bn(zlv, glv_ref[:], betalv_ref[:]))


def _mlp(pooled, read_depth, W1a, w1b, b1, W2, b2, Wmu, bmu, Wlv, blv,
         gmu, betamu, glv, betalv):
    return pl.pallas_call(
        _mlp_body,
        out_shape=(jax.ShapeDtypeStruct((B, NUM_TOPICS), jnp.float32),
                   jax.ShapeDtypeStruct((B, NUM_TOPICS), jnp.float32)),
    )(pooled, read_depth, W1a, w1b, b1, W2, b2, Wmu, bmu, Wlv, blv,
      gmu, betamu, glv, betalv)


def kernel(idx, read_depth, table, W1, b1, W2, b2, Wmu, bmu, Wlv, blv,
           gmu, betamu, glv, betalv):
    table_lin = _relayout(jnp.transpose(table)).reshape(NP2, HIDDEN)
    idxT = jnp.transpose(idx.astype(jnp.int32))
    pooled = _sc_pool(idxT, table_lin)
    W1a = W1[:, :HIDDEN]
    w1b = W1[:, HIDDEN].reshape(1, HIDDEN)
    return _mlp(pooled, read_depth, W1a, w1b, b1.reshape(1, HIDDEN),
                W2, b2.reshape(1, HIDDEN), Wmu, bmu.reshape(1, NUM_TOPICS),
                Wlv, blv.reshape(1, NUM_TOPICS), gmu.reshape(1, NUM_TOPICS),
                betamu.reshape(1, NUM_TOPICS), glv.reshape(1, NUM_TOPICS),
                betalv.reshape(1, NUM_TOPICS))


# speedup vs baseline: 2.9856x; 2.9856x over previous
import functools

import jax
import jax.numpy as jnp
from jax import lax
from jax.experimental import pallas as pl
from jax.experimental.pallas import tpu as pltpu
from jax.experimental.pallas import tpu_sc as plsc

NUM_PEAKS = 1000000
HIDDEN = 64
NUM_TOPICS = 32
B = 4096
L = 200

NC = 2
NS = 16
NW = NC * NS
PER_W = B // NW
CHUNK = 100
NCHUNK = L // CHUNK


NBUF = 4
UN = 8
NG = HIDDEN // 16


def _sc_pool_body(idx_hbm, table_hbm, out_hbm, idx_v, rows_v, pool_v,
                  s0, s1, s2, s3):
    sems = (s0, s1, s2, s3)
    wid = lax.axis_index("s") * NC + lax.axis_index("c")
    pltpu.sync_copy(idx_hbm.at[pl.ds(wid * PER_W * NCHUNK, PER_W * NCHUNK)],
                    idx_v)

    def descs(e, j):
        return (
            pltpu.make_async_copy(table_hbm.at[idx_v.at[NCHUNK * e]],
                                  rows_v.at[j, pl.ds(0, CHUNK)], sems[j]),
            pltpu.make_async_copy(table_hbm.at[idx_v.at[NCHUNK * e + 1]],
                                  rows_v.at[j, pl.ds(CHUNK, CHUNK)], sems[j]),
        )

    def start(e, j):
        for d in descs(e, j):
            d.start()

    def wait(e, j):
        for d in descs(e, j):
            d.wait()

    def reduce_into(j, b):
        def red(r, acc):
            accs = list(acc)
            for u in range(UN):
                for g in range(NG):
                    a = 2 * g + (u & 1)
                    accs[a] = accs[a] + rows_v[j, r * UN + u,
                                               pl.ds(16 * g, 16)]
            return tuple(accs)

        acc = lax.fori_loop(0, L // UN, red,
                            tuple(jnp.zeros((16,), jnp.float32)
                                  for _ in range(2 * NG)))
        for g in range(NG):
            pool_v[b, pl.ds(16 * g, 16)] = acc[2 * g] + acc[2 * g + 1]

    for j in range(NBUF):
        start(j, j)

    def outer(i, _):
        k = i * NBUF
        for j in range(NBUF):
            e = k + j
            wait(e, j)
            reduce_into(j, e)

            @pl.when(e + NBUF < PER_W)
            def _():
                start(e + NBUF, j)
        return 0

    lax.fori_loop(0, PER_W // NBUF, outer, 0)
    pltpu.sync_copy(pool_v, out_hbm.at[pl.ds(wid * PER_W, PER_W)])


_sc_pool = functools.partial(
    pl.kernel,
    mesh=plsc.VectorSubcoreMesh(core_axis_name="c", subcore_axis_name="s"),
    out_type=jax.ShapeDtypeStruct((B, HIDDEN), jnp.float32),
    scratch_types=[
        pltpu.VMEM((PER_W * NCHUNK, CHUNK), jnp.int32),
        pltpu.VMEM((NBUF, L, HIDDEN), jnp.float32),
        pltpu.VMEM((PER_W, HIDDEN), jnp.float32),
        pltpu.SemaphoreType.DMA,
        pltpu.SemaphoreType.DMA,
        pltpu.SemaphoreType.DMA,
        pltpu.SemaphoreType.DMA,
    ],
    compiler_params=pltpu.CompilerParams(use_tc_tiling_on_sc=False),
)(_sc_pool_body)


TBLK = 32768
TGRID = -(-NUM_PEAKS // TBLK)
THALF = TBLK // 2
NP2 = TGRID * TBLK


def _relayout_body(tT_ref, out_ref):
    out_ref[:, 0:HIDDEN] = jnp.transpose(tT_ref[:, 0:THALF])
    out_ref[:, HIDDEN:2 * HIDDEN] = jnp.transpose(tT_ref[:, THALF:TBLK])


def _relayout(tableT):
    return pl.pallas_call(
        _relayout_body,
        grid=(TGRID,),
        in_specs=[pl.BlockSpec((HIDDEN, TBLK), lambda i: (0, i))],
        out_specs=pl.BlockSpec((THALF, 2 * HIDDEN), lambda i: (i, 0)),
        out_shape=jax.ShapeDtypeStruct((NP2 // 2, 2 * HIDDEN),
                                       jnp.float32),
    )(tableT)


def _softplus(x):
    return jnp.maximum(x, 0.0) + jnp.log1p(jnp.exp(-jnp.abs(x)))


def _mlp_body(pooled_ref, rd_ref, W1a_ref, w1b_ref, b1_ref, W2_ref, b2_ref,
              Wmu_ref, bmu_ref, Wlv_ref, blv_ref, gmu_ref, betamu_ref,
              glv_ref, betalv_ref, loc_ref, scale_ref):
    rd = rd_ref[:]
    ave = pooled_ref[:] / rd
    lrd = jnp.log(rd)
    z1 = (lax.dot_general(ave, W1a_ref[:], (((1,), (1,)), ((), ())),
                          preferred_element_type=jnp.float32)
          + lrd * w1b_ref[:] + b1_ref[:])
    h1 = _softplus(z1)
    z2 = (lax.dot_general(h1, W2_ref[:], (((1,), (1,)), ((), ())),
                          preferred_element_type=jnp.float32)
          + b2_ref[:])
    h2 = _softplus(z2)

    def bn(z, g, bt):
        m = jnp.mean(z, axis=0)
        v = jnp.mean((z - m) ** 2, axis=0)
        return (z - m) / jnp.sqrt(v + 1e-5) * g + bt

    zmu = (lax.dot_general(h2, Wmu_ref[:], (((1,), (1,)), ((), ())),
                           preferred_element_type=jnp.float32)
           + bmu_ref[:])
    zlv = (lax.dot_general(h2, Wlv_ref[:], (((1,), (1,)), ((), ())),
                           preferred_element_type=jnp.float32)
           + blv_ref[:])
    loc_ref[:] = bn(zmu, gmu_ref[:], betamu_ref[:])
    scale_ref[:] = jnp.exp(0.5 * bn(zlv, glv_ref[:], betalv_ref[:]))


def _mlp(pooled, read_depth, W1a, w1b, b1, W2, b2, Wmu, bmu, Wlv, blv,
         gmu, betamu, glv, betalv):
    return pl.pallas_call(
        _mlp_body,
        out_shape=(jax.ShapeDtypeStruct((B, NUM_TOPICS), jnp.float32),
                   jax.ShapeDtypeStruct((B, NUM_TOPICS), jnp.float32)),
    )(pooled, read_depth, W1a, w1b, b1, W2, b2, Wmu, bmu, Wlv, blv,
      gmu, betamu, glv, betalv)


def kernel(idx, read_depth, table, W1, b1, W2, b2, Wmu, bmu, Wlv, blv,
           gmu, betamu, glv, betalv):
    table_lin = _relayout(jnp.transpose(table)).reshape(NP2, HIDDEN)
    r = idx.astype(jnp.int32)
    q = r % TBLK
    g = (r - q) + 2 * (q % THALF) + (q >= THALF).astype(jnp.int32)
    idx2 = g.reshape(B * NCHUNK, CHUNK)
    pooled = _sc_pool(idx2, table_lin)
    W1a = W1[:, :HIDDEN]
    w1b = W1[:, HIDDEN].reshape(1, HIDDEN)
    return _mlp(pooled, read_depth, W1a, w1b, b1.reshape(1, HIDDEN),
                W2, b2.reshape(1, HIDDEN), Wmu, bmu.reshape(1, NUM_TOPICS),
                Wlv, blv.reshape(1, NUM_TOPICS), gmu.reshape(1, NUM_TOPICS),
                betamu.reshape(1, NUM_TOPICS), glv.reshape(1, NUM_TOPICS),
                betalv.reshape(1, NUM_TOPICS))
